# baseline (speedup 1.0000x reference)
import functools

import jax
import jax.numpy as jnp
from jax import lax
from jax.experimental import pallas as pl
from jax.experimental.pallas import tpu as pltpu
from jax.experimental.pallas import tpu_sc as plsc

NBUF = 8
PSEQ = 56
NC = 2
NS = 16
NW = NC * NS


def _embed_body(bpw, seq_len, emb, seq_hbm, table_hbm, out_hbm, idx_v, bufs,
                *sems):
    gsems = sems[:NBUF]
    osems = sems[NBUF:]
    c = lax.axis_index("c")
    s = lax.axis_index("s")
    wid = s * NC + c
    b0 = wid * bpw

    pltpu.sync_copy(seq_hbm.at[pl.ds(b0, bpw)], idx_v)

    def gather(j, slot):
        return pltpu.make_async_copy(
            table_hbm.at[idx_v.at[j]], bufs.at[slot], gsems[slot]
        )

    for slot in range(NBUF):
        gather(slot, slot).start()

    def one_round(i, refill):
        writes = []
        for slot in range(NBUF):
            j = i * NBUF + slot
            gather(j, slot).wait()
            writes.append(
                pltpu.async_copy(bufs.at[slot].at[pl.ds(0, seq_len)],
                                 out_hbm.at[b0 + j], osems[slot])
            )
        for slot in range(NBUF):
            writes[slot].wait()
            if refill:
                gather((i + 1) * NBUF + slot, slot).start()

    n_rounds = bpw // NBUF
    lax.fori_loop(0, n_rounds - 1, lambda i, _: (one_round(i, True), 0)[1], 0)
    one_round(n_rounds - 1, False)


@functools.partial(jax.jit, static_argnums=(2,))
def _embed_call(seq_padded, table, seq_len):
    batch = seq_padded.shape[0]
    emb = table.shape[1]
    bpw = batch // NW
    grid_kernel = pl.kernel(
        functools.partial(_embed_body, bpw, seq_len, emb),
        out_type=jax.ShapeDtypeStruct((batch, seq_len, emb), jnp.float32),
        mesh=plsc.VectorSubcoreMesh(
            core_axis_name="c", subcore_axis_name="s",
            num_cores=NC, num_subcores=NS,
        ),
        scratch_types=[
            pltpu.VMEM((bpw, PSEQ), jnp.int32),
            pltpu.VMEM((NBUF, PSEQ, emb), jnp.float32),
        ] + [pltpu.SemaphoreType.DMA] * (2 * NBUF),
        compiler_params=pltpu.CompilerParams(use_tc_tiling_on_sc=False),
    )
    return grid_kernel(seq_padded, table)


def kernel(seq, table):
    batch, seq_len = seq.shape
    assert batch % (NW * NBUF) == 0 and seq_len <= PSEQ
    seq_padded = jnp.pad(seq.astype(jnp.int32), ((0, 0), (0, PSEQ - seq_len)))
    return _embed_call(seq_padded, table, seq_len)


# speedup vs baseline: 1.6879x; 1.6879x over previous
import functools

import jax
import jax.numpy as jnp
from jax import lax
from jax.experimental import pallas as pl
from jax.experimental.pallas import tpu as pltpu
from jax.experimental.pallas import tpu_sc as plsc

NBUF = 10
CHUNK = 64
NC = 2
NS = 16
NW = NC * NS
L = 16


def _embed_body(lpw, seq_hbm, table_hbm, out_hbm, idx_v, idx2, bufs, *sems):
    gsems = sems[:NBUF]
    osems = sems[NBUF:]
    c = lax.axis_index("c")
    s = lax.axis_index("s")
    wid = s * NC + c

    pltpu.sync_copy(seq_hbm.at[wid], idx_v)

    iota = lax.iota(jnp.int32, L)

    def build(g, _):
        v2 = idx_v[g] * 2
        pos = g * (2 * L) + iota * 2
        plsc.store_scatter(idx2, [pos], v2)
        plsc.store_scatter(idx2, [pos + 1], v2 + 1)
        return 0

    lax.fori_loop(0, lpw // L, build, 0)

    def gather(j, slot):
        return pltpu.make_async_copy(
            table_hbm.at[idx2.at[pl.ds(j * 2 * CHUNK, 2 * CHUNK)]],
            bufs.at[slot],
            gsems[slot],
        )

    for slot in range(NBUF):
        gather(slot, slot).start()

    out0 = wid * 2 * lpw

    def one_round(i, refill):
        writes = []
        for slot in range(NBUF):
            j = i * NBUF + slot
            gather(j, slot).wait()
            writes.append(
                pltpu.async_copy(
                    bufs.at[slot],
                    out_hbm.at[pl.ds(out0 + j * 2 * CHUNK, 2 * CHUNK)],
                    osems[slot],
                )
            )
        for slot in range(NBUF):
            writes[slot].wait()
            if refill:
                gather((i + 1) * NBUF + slot, slot).start()

    n_rounds = lpw // (CHUNK * NBUF)
    lax.fori_loop(0, n_rounds - 1, lambda i, _: (one_round(i, True), 0)[1], 0)
    one_round(n_rounds - 1, False)


@jax.jit
def _embed_call(seq3d, table2):
    nw, rows, lanes = seq3d.shape
    lpw = rows * lanes
    grid_kernel = pl.kernel(
        functools.partial(_embed_body, lpw),
        out_type=jax.ShapeDtypeStruct((NW * lpw * 2, 32), jnp.float32),
        mesh=plsc.VectorSubcoreMesh(
            core_axis_name="c", subcore_axis_name="s",
            num_cores=NC, num_subcores=NS,
        ),
        scratch_types=[
            pltpu.VMEM((rows, lanes), jnp.int32),
            pltpu.VMEM((lpw * 2,), jnp.int32),
            pltpu.VMEM((NBUF, 2 * CHUNK, 32), jnp.float32),
        ] + [pltpu.SemaphoreType.DMA] * (2 * NBUF),
        compiler_params=pltpu.CompilerParams(
            use_tc_tiling_on_sc=False, needs_layout_passes=False
        ),
    )
    return grid_kernel(seq3d, table2)


def kernel(seq, table):
    batch, seq_len = seq.shape
    total = batch * seq_len
    assert total % (NW * CHUNK * NBUF) == 0
    seq3d = seq.reshape(NW, total // (NW * L), L).astype(jnp.int32)
    table2 = table.reshape(table.shape[0] * 2, 32)
    out = _embed_call(seq3d, table2)
    return out.reshape(batch, seq_len, table.shape[1])
